# baseline (speedup 1.0000x reference)
import functools

import jax
import jax.numpy as jnp
from jax import lax
from jax.experimental import pallas as pl
from jax.experimental.pallas import tpu as pltpu
from jax.experimental.pallas import tpu_sc as plsc

NC = 2
NS = 16
NW = NC * NS
L = 16
Q = 4
SR = 16
STRIDE = 5


def _rsqrt(x):
    i = plsc.bitcast(x, jnp.int32)
    i = jnp.int32(0x5F3759DF) - lax.shift_right_arithmetic(i, 1)
    y = plsc.bitcast(i, jnp.float32)
    for _ in range(3):
        y = y * (jnp.float32(1.5) - jnp.float32(0.5) * x * y * y)
    return y


def _sqrt(x):
    return x * _rsqrt(x)


def _log(z):
    zi = plsc.bitcast(z, jnp.int32)
    ex = lax.shift_right_arithmetic(zi, 23) - jnp.int32(127)
    mi = (zi & jnp.int32(0x007FFFFF)) | jnp.int32(0x3F800000)
    m = plsc.bitcast(mi, jnp.float32)
    big = m > jnp.float32(1.4142135)
    m = jnp.where(big, m * jnp.float32(0.5), m)
    ex = ex + jnp.where(big, jnp.int32(1), jnp.int32(0))
    s = (m - jnp.float32(1.0)) / (m + jnp.float32(1.0))
    s2 = s * s
    p = s2 * jnp.float32(1.0 / 9.0)
    for c in (1.0 / 7.0, 1.0 / 5.0, 1.0 / 3.0, 1.0):
        p = s2 * p + jnp.float32(c)
    p = jnp.float32(2.0) * s * p
    return ex.astype(jnp.float32) * jnp.float32(0.6931471805599453) + p


def _make_sc_kernel(N, D, B):
    assert D == 64 and B % NW == 0
    bpw = B // NW
    ngrp = bpw // L
    nch = bpw // 128
    dh = D // 2
    mesh = plsc.VectorSubcoreMesh(core_axis_name="c", subcore_axis_name="s",
                                  num_cores=NC, num_subcores=NS)

    @functools.partial(
        pl.kernel,
        mesh=mesh,
        out_type=jax.ShapeDtypeStruct((B,), jnp.float32),
        compiler_params=pltpu.CompilerParams(needs_layout_passes=False,
                                             use_tc_tiling_on_sc=False),
        scratch_types=[
            pltpu.VMEM((dh, L), jnp.float32),
            pltpu.VMEM((dh, L), jnp.float32),
            pltpu.VMEM((bpw,), jnp.int32),
            pltpu.VMEM((bpw,), jnp.int32),
            pltpu.VMEM((Q, bpw), jnp.int32),
            pltpu.VMEM((Q, bpw), jnp.int32),
            pltpu.VMEM((Q, bpw, SR), jnp.float32),
            pltpu.VMEM((Q, bpw, SR), jnp.float32),
            pltpu.VMEM((bpw,), jnp.float32),
            pltpu.VMEM((bpw,), jnp.float32),
            pltpu.VMEM((bpw,), jnp.float32),
            pltpu.VMEM((bpw,), jnp.float32),
            pltpu.SemaphoreType.DMA,
        ],
    )
    def sc_kernel(cb_hbm, sb_hbm, uidx_hbm, vidx_hbm, w_hbm, tab_hbm,
                  bias_hbm, out_hbm, cb_v, sb_v, uidx_v, vidx_v, u5_v, v5_v,
                  ru, rv, w_v, bu_v, bv_v, out_v, sem):
        wid = lax.axis_index("s") * NC + lax.axis_index("c")
        base = wid * bpw
        pltpu.sync_copy(cb_hbm, cb_v)
        pltpu.sync_copy(sb_hbm, sb_v)
        pltpu.sync_copy(uidx_hbm.at[pl.ds(base, bpw)], uidx_v)
        pltpu.sync_copy(vidx_hbm.at[pl.ds(base, bpw)], vidx_v)
        pltpu.sync_copy(w_hbm.at[pl.ds(base, bpw)], w_v)

        def rowids(g, carry):
            sl = pl.ds(g * L, L)
            iu = uidx_v[sl] * jnp.int32(STRIDE)
            iv = vidx_v[sl] * jnp.int32(STRIDE)
            for q in range(Q):
                u5_v[q, sl] = iu + jnp.int32(q)
                v5_v[q, sl] = iv + jnp.int32(q)
            return carry

        lax.fori_loop(0, ngrp, rowids, 0)

        cps = []
        for k in range(nch):
            sl = pl.ds(k * 128, 128)
            cps.append(pltpu.async_copy(
                bias_hbm.at[uidx_v.at[sl]], bu_v.at[sl], sem))
            cps.append(pltpu.async_copy(
                bias_hbm.at[vidx_v.at[sl]], bv_v.at[sl], sem))

        def fire(k, carry):
            sl = pl.ds(k * 128, 128)
            for q in range(Q):
                pltpu.async_copy(tab_hbm.at[u5_v.at[q].at[sl]],
                                 ru.at[q].at[sl], sem)
                pltpu.async_copy(tab_hbm.at[v5_v.at[q].at[sl]],
                                 rv.at[q].at[sl], sem)
            return carry

        lax.fori_loop(0, nch, fire, 0)
        for q in range(Q):
            pltpu.make_async_copy(tab_hbm.at[pl.ds(0, bpw)], ru.at[q],
                                  sem).wait()
            pltpu.make_async_copy(tab_hbm.at[pl.ds(0, bpw)], rv.at[q],
                                  sem).wait()
        pltpu.make_async_copy(tab_hbm.at[pl.ds(0, bpw // SR)], bu_v,
                              sem).wait()
        pltpu.make_async_copy(tab_hbm.at[pl.ds(0, bpw // SR)], bv_v,
                              sem).wait()

        iota16 = lax.iota(jnp.int32, L)

        def group(g, carry):
            p0 = g * L
            psl = pl.ds(p0, L)
            idx_p = p0 + iota16
            nu = jnp.zeros((L,), jnp.float32)
            nv = jnp.zeros((L,), jnp.float32)
            dot = jnp.zeros((L,), jnp.float32)
            for j in range(dh):
                q = (2 * j) // SR
                ce = jnp.full((L,), (2 * j) % SR, jnp.int32)
                co = jnp.full((L,), (2 * j + 1) % SR, jnp.int32)
                ue = plsc.load_gather(ru.at[q], [idx_p, ce])
                uo = plsc.load_gather(ru.at[q], [idx_p, co])
                ve = plsc.load_gather(rv.at[q], [idx_p, ce])
                vo = plsc.load_gather(rv.at[q], [idx_p, co])
                cj = cb_v[j, :]
                sj = sb_v[j, :]
                nu = nu + (ue * ue + uo * uo)
                nv = nv + (ve * ve + vo * vo)
                dot = dot + cj * (ue * ve + uo * vo) + sj * (uo * ve - ue * vo)
            x0u = _sqrt(jnp.float32(1.0) + nu)
            x0v = _sqrt(jnp.float32(1.0) + nv)
            minner = x0u * x0v - dot
            arg = jnp.maximum(minner, jnp.float32(1.0 + 1e-7))
            e = arg - jnp.float32(1.0)
            t = e + _sqrt(e * (e + jnp.float32(2.0)))
            d = _log(jnp.float32(1.0) + t)
            wv = w_v[psl]
            out_v[psl] = -wv * d * d + bu_v[psl] + bv_v[psl]
            return carry

        lax.fori_loop(0, ngrp, group, 0)
        pltpu.sync_copy(out_v, out_hbm.at[pl.ds(base, bpw)])

    return sc_kernel


def kernel(u_idx, v_idx, w_uv, theta_src, theta_dst, eucl, bias):
    N, D = eucl.shape
    B = u_idx.shape[0]
    phi = theta_dst - theta_src
    cb = jnp.broadcast_to(jnp.cos(phi)[:, None], (D // 2, L))
    sb = jnp.broadcast_to(jnp.sin(phi)[:, None], (D // 2, L))
    tab = jnp.pad(eucl, ((0, 0), (0, STRIDE * SR - D))).reshape(
        STRIDE * N, SR)
    sc = _make_sc_kernel(N, D, B)
    return sc(cb.astype(jnp.float32), sb.astype(jnp.float32),
              u_idx.astype(jnp.int32), v_idx.astype(jnp.int32),
              w_uv, tab, bias)
